# Initial kernel scaffold; baseline (speedup 1.0000x reference)
import jax
import jax.numpy as jnp
from jax.experimental import pallas as pl


def kernel(x, pos, edge_index, edge_attr, params):
    raise NotImplementedError("write your pallas kernel here")


# speedup vs baseline: 3.8813x; 3.8813x over previous
import functools

import jax
import jax.numpy as jnp
from jax import lax
from jax.experimental import pallas as pl
from jax.experimental.pallas import tpu as pltpu
from jax.experimental.pallas import tpu_sc as plsc

N = 10000
E = 640000
D = 64
H = 128
NC = 2
NS = 16
NW = NC * NS
EPW = E // NW
CH = 80
NCH = EPW // CH
NP = 10240
RPT = NP // NS
BE = 1024
GE = E // BE
BN = 1000
GN = N // BN


def _sc_gather_body(h_hbm, pos_hbm, src_hbm, dst_hbm,
                    hs_out, hd_out, ps_out, pd_out,
                    idx_s, idx_d, hbuf_s, hbuf_d, pbuf_s, pbuf_d, sem):
    c = lax.axis_index("c")
    s = lax.axis_index("s")
    wid = s * NC + c
    pltpu.sync_copy(src_hbm.at[wid], idx_s)
    pltpu.sync_copy(dst_hbm.at[wid], idx_d)

    def body(g, carry):
        e0 = wid * EPW + g * CH
        cps = [
            pltpu.make_async_copy(h_hbm.at[idx_s.at[g]], hbuf_s, sem),
            pltpu.make_async_copy(h_hbm.at[idx_d.at[g]], hbuf_d, sem),
            pltpu.make_async_copy(pos_hbm.at[idx_s.at[g]], pbuf_s, sem),
            pltpu.make_async_copy(pos_hbm.at[idx_d.at[g]], pbuf_d, sem),
        ]
        for cp in cps:
            cp.start()
        for cp in cps:
            cp.wait()
        pltpu.sync_copy(hbuf_s, hs_out.at[pl.ds(e0, CH)])
        pltpu.sync_copy(hbuf_d, hd_out.at[pl.ds(e0, CH)])
        pltpu.sync_copy(pbuf_s, ps_out.at[pl.ds(e0, CH)])
        pltpu.sync_copy(pbuf_d, pd_out.at[pl.ds(e0, CH)])
        return carry

    lax.fori_loop(0, NCH, body, 0)


def _sc_scatter_body(with_w, *refs):
    if with_w:
        (m2_hbm, wd_hbm, dst_hbm, zm_hbm, za_hbm, outm, outa,
         idxv, m2v, wdv, macc, aacc) = refs
    else:
        (m2_hbm, dst_hbm, zm_hbm, outm, idxv, m2v, macc) = refs
    c = lax.axis_index("c")
    s = lax.axis_index("s")
    wid = s * NC + c
    pltpu.sync_copy(zm_hbm, macc.at[pl.ds(s * RPT, RPT)])
    if with_w:
        pltpu.sync_copy(za_hbm, aacc.at[pl.ds(s * RPT, RPT)])
    pltpu.sync_copy(dst_hbm.at[wid], idxv)
    plsc.subcore_barrier()

    def body(g, carry):
        e0 = wid * EPW + g * CH
        pltpu.sync_copy(m2_hbm.at[pl.ds(e0, CH)], m2v)
        pltpu.sync_copy(m2v, macc.at[idxv.at[g]], add=True)
        if with_w:
            pltpu.sync_copy(wd_hbm.at[pl.ds(e0, CH)], wdv)
            pltpu.sync_copy(wdv, aacc.at[idxv.at[g]], add=True)
        return carry

    lax.fori_loop(0, NCH, body, 0)
    plsc.subcore_barrier()
    pltpu.sync_copy(macc.at[pl.ds(s * RPT, RPT)],
                    outm.at[c].at[pl.ds(s * RPT, RPT)])
    if with_w:
        pltpu.sync_copy(aacc.at[pl.ds(s * RPT, RPT)],
                        outa.at[c].at[pl.ds(s * RPT, RPT)])


@functools.lru_cache(maxsize=None)
def _sc_mesh():
    return plsc.VectorSubcoreMesh(core_axis_name="c", subcore_axis_name="s")


@functools.lru_cache(maxsize=None)
def _sc_gather():
    return pl.kernel(
        _sc_gather_body,
        mesh=_sc_mesh(),
        out_type=[
            jax.ShapeDtypeStruct((E, D), jnp.float32),
            jax.ShapeDtypeStruct((E, D), jnp.float32),
            jax.ShapeDtypeStruct((E, 16), jnp.float32),
            jax.ShapeDtypeStruct((E, 16), jnp.float32),
        ],
        scratch_types=[
            pltpu.VMEM((NCH, CH), jnp.int32),
            pltpu.VMEM((NCH, CH), jnp.int32),
            pltpu.VMEM((CH, D), jnp.float32),
            pltpu.VMEM((CH, D), jnp.float32),
            pltpu.VMEM((CH, 16), jnp.float32),
            pltpu.VMEM((CH, 16), jnp.float32),
            pltpu.SemaphoreType.DMA,
        ],
        compiler_params=pltpu.CompilerParams(use_tc_tiling_on_sc=False),
    )


@functools.lru_cache(maxsize=None)
def _sc_scatter(with_w):
    out_type = [jax.ShapeDtypeStruct((NC, NP, H), jnp.float32)]
    scratch = [
        pltpu.VMEM((NCH, CH), jnp.int32),
        pltpu.VMEM((CH, H), jnp.float32),
    ]
    if with_w:
        out_type.append(jax.ShapeDtypeStruct((NC, NP, 16), jnp.float32))
        scratch.append(pltpu.VMEM((CH, 16), jnp.float32))
    scratch.append(pltpu.VMEM_SHARED((NP, H), jnp.float32))
    if with_w:
        scratch.append(pltpu.VMEM_SHARED((NP, 16), jnp.float32))
    return pl.kernel(
        functools.partial(_sc_scatter_body, with_w),
        mesh=_sc_mesh(),
        out_type=out_type,
        scratch_types=scratch,
        compiler_params=pltpu.CompilerParams(use_tc_tiling_on_sc=False),
    )


def _edge_body(with_w, hs, hd, ps, pd, ea,
               whd, whs, wea, wd2, be1, we2, be2, wx, bx, *outs):
    diff = pd[...] - ps[...]
    dist2 = jnp.sum(diff * diff, axis=1, keepdims=True)
    a = jnp.dot(hd[...], whd[...], preferred_element_type=jnp.float32)
    a = a + jnp.dot(hs[...], whs[...], preferred_element_type=jnp.float32)
    a = a + jnp.dot(ea[...], wea[...], preferred_element_type=jnp.float32)
    a = a + dist2 * wd2[...] + be1[...]
    m1 = a * jax.nn.sigmoid(a)
    b = jnp.dot(m1, we2[...], preferred_element_type=jnp.float32) + be2[...]
    m2 = b * jax.nn.sigmoid(b)
    outs[0][...] = m2
    if with_w:
        wv = jnp.tanh(jnp.sum(m2 * wx[...], axis=1, keepdims=True)
                      + bx[...][:, :1])
        wd = diff * wv
        col = lax.broadcasted_iota(jnp.int32, wd.shape, 1)
        outs[1][...] = jnp.where(col == 3, 1.0, wd)


def _edge_call(with_w):
    w_specs = [
        pl.BlockSpec((D, H), lambda i: (0, 0)),
        pl.BlockSpec((D, H), lambda i: (0, 0)),
        pl.BlockSpec((16, H), lambda i: (0, 0)),
        pl.BlockSpec((1, H), lambda i: (0, 0)),
        pl.BlockSpec((1, H), lambda i: (0, 0)),
        pl.BlockSpec((H, H), lambda i: (0, 0)),
        pl.BlockSpec((1, H), lambda i: (0, 0)),
        pl.BlockSpec((1, H), lambda i: (0, 0)),
        pl.BlockSpec((1, H), lambda i: (0, 0)),
    ]
    out_shape = [jax.ShapeDtypeStruct((E, H), jnp.float32)]
    out_specs = [pl.BlockSpec((BE, H), lambda i: (i, 0))]
    if with_w:
        out_shape.append(jax.ShapeDtypeStruct((E, 16), jnp.float32))
        out_specs.append(pl.BlockSpec((BE, 16), lambda i: (i, 0)))
    return pl.pallas_call(
        functools.partial(_edge_body, with_w),
        grid=(GE,),
        in_specs=[
            pl.BlockSpec((BE, D), lambda i: (i, 0)),
            pl.BlockSpec((BE, D), lambda i: (i, 0)),
            pl.BlockSpec((BE, 16), lambda i: (i, 0)),
            pl.BlockSpec((BE, 16), lambda i: (i, 0)),
            pl.BlockSpec((BE, 16), lambda i: (i, 0)),
        ] + w_specs,
        out_specs=out_specs,
        out_shape=out_shape,
        compiler_params=pltpu.CompilerParams(
            dimension_semantics=("arbitrary",)),
    )


def _node_body(with_c, xb, m0, m1, a0, a1, pb,
               wh1a, wh1b, bh1, wh2, bh2, *outs):
    magg = m0[...] + m1[...]
    t = (jnp.dot(xb[...], wh1a[...], preferred_element_type=jnp.float32)
         + jnp.dot(magg, wh1b[...], preferred_element_type=jnp.float32)
         + bh1[...])
    t = t * jax.nn.sigmoid(t)
    dh = jnp.dot(t, wh2[...], preferred_element_type=jnp.float32) + bh2[...]
    outs[0][...] = xb[...] + dh
    if with_c:
        ax = a0[...] + a1[...]
        cnt = jnp.maximum(ax[:, 3:4], 1.0)
        upd = ax / cnt
        col = lax.broadcasted_iota(jnp.int32, upd.shape, 1)
        outs[1][...] = pb[...] + jnp.where(col < 3, upd, 0.0)


def _node_call(with_c):
    in_specs = [
        pl.BlockSpec((BN, D), lambda i: (i, 0)),
        pl.BlockSpec((BN, H), lambda i: (i, 0)),
        pl.BlockSpec((BN, H), lambda i: (i, 0)),
        pl.BlockSpec((BN, 16), lambda i: (i, 0)),
        pl.BlockSpec((BN, 16), lambda i: (i, 0)),
        pl.BlockSpec((BN, 16), lambda i: (i, 0)),
        pl.BlockSpec((D, H), lambda i: (0, 0)),
        pl.BlockSpec((H, H), lambda i: (0, 0)),
        pl.BlockSpec((1, H), lambda i: (0, 0)),
        pl.BlockSpec((H, D), lambda i: (0, 0)),
        pl.BlockSpec((1, D), lambda i: (0, 0)),
    ]
    out_shape = [jax.ShapeDtypeStruct((N, D), jnp.float32)]
    out_specs = [pl.BlockSpec((BN, D), lambda i: (i, 0))]
    if with_c:
        out_shape.append(jax.ShapeDtypeStruct((N, 16), jnp.float32))
        out_specs.append(pl.BlockSpec((BN, 16), lambda i: (i, 0)))
    return pl.pallas_call(
        functools.partial(_node_body, with_c),
        grid=(GN,),
        in_specs=in_specs,
        out_specs=out_specs,
        out_shape=out_shape,
        compiler_params=pltpu.CompilerParams(
            dimension_semantics=("arbitrary",)),
    )


def _layer_weights(lp):
    we1 = lp["We1"]
    return dict(
        whd=we1[0:D],
        whs=we1[D:2 * D],
        wd2=we1[2 * D:2 * D + 1],
        wea=jnp.pad(we1[2 * D + 1:], ((0, 1), (0, 0))),
        be1=lp["be1"][None, :],
        we2=lp["We2"],
        be2=lp["be2"][None, :],
        wx=lp["Wx"].T,
        bx=jnp.broadcast_to(lp["bx"].reshape(1, 1), (1, H)),
        wh1a=lp["Wh1"][0:D],
        wh1b=lp["Wh1"][D:],
        bh1=lp["bh1"][None, :],
        wh2=lp["Wh2"],
        bh2=lp["bh2"][None, :],
    )


def kernel(x, pos, edge_index, edge_attr, params):
    src2d = edge_index[0].reshape(NW, NCH, CH)
    dst2d = edge_index[1].reshape(NW, NCH, CH)
    pos_pad = jnp.pad(pos, ((0, 0), (0, 13)))
    ea_pad = jnp.pad(edge_attr, ((0, 0), (0, 1)))
    zm = jnp.zeros((RPT, H), jnp.float32)
    za = jnp.zeros((RPT, 16), jnp.float32)
    w1 = _layer_weights(params["layers"][0])
    w2 = _layer_weights(params["layers"][1])

    hs, hd, ps, pd = _sc_gather()(x, pos_pad, src2d, dst2d)
    m2, wd = _edge_call(True)(
        hs, hd, ps, pd, ea_pad,
        w1["whd"], w1["whs"], w1["wea"], w1["wd2"], w1["be1"],
        w1["we2"], w1["be2"], w1["wx"], w1["bx"])
    mparts, aparts = _sc_scatter(True)(m2, wd, dst2d, zm, za)
    h1, c1 = _node_call(True)(
        x, mparts[0], mparts[1], aparts[0], aparts[1], pos_pad,
        w1["wh1a"], w1["wh1b"], w1["bh1"], w1["wh2"], w1["bh2"])

    hs2, hd2, ps2, pd2 = _sc_gather()(h1, c1, src2d, dst2d)
    m2b, = _edge_call(False)(
        hs2, hd2, ps2, pd2, ea_pad,
        w2["whd"], w2["whs"], w2["wea"], w2["wd2"], w2["be1"],
        w2["we2"], w2["be2"], w2["wx"], w2["bx"])
    mparts2, = _sc_scatter(False)(m2b, dst2d, zm)
    h2, = _node_call(False)(
        h1, mparts2[0], mparts2[1], aparts[0], aparts[1], pos_pad,
        w2["wh1a"], w2["wh1b"], w2["bh1"], w2["wh2"], w2["bh2"])
    return h2


# speedup vs baseline: 5.1562x; 1.3285x over previous
import functools

import jax
import jax.numpy as jnp
from jax import lax
from jax.experimental import pallas as pl
from jax.experimental.pallas import tpu as pltpu
from jax.experimental.pallas import tpu_sc as plsc

N = 10000
E = 640000
D = 64
DP = 80
H = 128
NC = 2
NS = 16
NW = NC * NS
EPW = E // NW
CH = 80
NCH = EPW // CH
NP = 10240
RPT = NP // NS
BE = 1024
GE = E // BE
BN = 1000
GN = N // BN

_SC_PARAMS = None


def _sc_gather_body(tab_hbm, src_hbm, dst_hbm, hps_out, hpd_out,
                    idx_s, idx_d, bs0, bd0, bs1, bd1, sem0, sem1):
    c = lax.axis_index("c")
    s = lax.axis_index("s")
    wid = s * NC + c
    pltpu.sync_copy(src_hbm.at[wid], idx_s)
    pltpu.sync_copy(dst_hbm.at[wid], idx_d)
    bufs = ((bs0, bd0, sem0), (bs1, bd1, sem1))

    def start(g, b):
        bs, bd, sem = bufs[b]
        pltpu.make_async_copy(tab_hbm.at[idx_s.at[g]], bs, sem).start()
        pltpu.make_async_copy(tab_hbm.at[idx_d.at[g]], bd, sem).start()

    def finish(g, b):
        bs, bd, sem = bufs[b]
        pltpu.make_async_copy(tab_hbm.at[idx_s.at[g]], bs, sem).wait()
        pltpu.make_async_copy(tab_hbm.at[idx_d.at[g]], bd, sem).wait()
        e0 = wid * EPW + g * CH
        pltpu.sync_copy(bs, hps_out.at[pl.ds(e0, CH)])
        pltpu.sync_copy(bd, hpd_out.at[pl.ds(e0, CH)])

    start(0, 0)

    def body(i, carry):
        g0 = 2 * i
        start(g0 + 1, 1)
        finish(g0, 0)

        @pl.when(i < NCH // 2 - 1)
        def _():
            start(g0 + 2, 0)

        finish(g0 + 1, 1)
        return carry

    lax.fori_loop(0, NCH // 2, body, 0)


def _sc_scatter_body(with_w, *refs):
    if with_w:
        (m2_hbm, wd_hbm, dst_hbm, zm_hbm, za_hbm, outm, outa,
         idx0, idx1, m0, m1, w0, w1, macc, aacc, sem0, sem1) = refs
        bufs = ((idx0, m0, w0, sem0), (idx1, m1, w1, sem1))
    else:
        (m2_hbm, dst_hbm, zm_hbm, outm, idx0, idx1, m0, m1,
         macc, sem0, sem1) = refs
        bufs = ((idx0, m0, None, sem0), (idx1, m1, None, sem1))
    c = lax.axis_index("c")
    s = lax.axis_index("s")
    wid = s * NC + c
    pltpu.sync_copy(zm_hbm, macc.at[pl.ds(s * RPT, RPT)])
    if with_w:
        pltpu.sync_copy(za_hbm, aacc.at[pl.ds(s * RPT, RPT)])
    plsc.subcore_barrier()

    def start(g, b):
        iv, mv, wv, sem = bufs[b]
        e0 = wid * EPW + g * CH
        pltpu.make_async_copy(dst_hbm.at[wid].at[pl.ds(g, 1)], iv, sem).start()
        pltpu.make_async_copy(m2_hbm.at[pl.ds(e0, CH)], mv, sem).start()
        if with_w:
            pltpu.make_async_copy(wd_hbm.at[pl.ds(e0, CH)], wv, sem).start()

    def finish(g, b):
        iv, mv, wv, sem = bufs[b]
        e0 = wid * EPW + g * CH
        pltpu.make_async_copy(dst_hbm.at[wid].at[pl.ds(g, 1)], iv, sem).wait()
        pltpu.make_async_copy(m2_hbm.at[pl.ds(e0, CH)], mv, sem).wait()
        if with_w:
            pltpu.make_async_copy(wd_hbm.at[pl.ds(e0, CH)], wv, sem).wait()
        pltpu.sync_copy(mv, macc.at[iv.at[0]], add=True)
        if with_w:
            pltpu.sync_copy(wv, aacc.at[iv.at[0]], add=True)

    start(0, 0)

    def body(i, carry):
        g0 = 2 * i
        start(g0 + 1, 1)
        finish(g0, 0)

        @pl.when(i < NCH // 2 - 1)
        def _():
            start(g0 + 2, 0)

        finish(g0 + 1, 1)
        return carry

    lax.fori_loop(0, NCH // 2, body, 0)
    plsc.subcore_barrier()
    pltpu.sync_copy(macc.at[pl.ds(s * RPT, RPT)],
                    outm.at[c].at[pl.ds(s * RPT, RPT)])
    if with_w:
        pltpu.sync_copy(aacc.at[pl.ds(s * RPT, RPT)],
                        outa.at[c].at[pl.ds(s * RPT, RPT)])


@functools.lru_cache(maxsize=None)
def _sc_mesh():
    return plsc.VectorSubcoreMesh(core_axis_name="c", subcore_axis_name="s")


@functools.lru_cache(maxsize=None)
def _sc_gather():
    return pl.kernel(
        _sc_gather_body,
        mesh=_sc_mesh(),
        out_type=[
            jax.ShapeDtypeStruct((E, DP), jnp.float32),
            jax.ShapeDtypeStruct((E, DP), jnp.float32),
        ],
        scratch_types=[
            pltpu.VMEM((NCH, CH), jnp.int32),
            pltpu.VMEM((NCH, CH), jnp.int32),
            pltpu.VMEM((CH, DP), jnp.float32),
            pltpu.VMEM((CH, DP), jnp.float32),
            pltpu.VMEM((CH, DP), jnp.float32),
            pltpu.VMEM((CH, DP), jnp.float32),
            pltpu.SemaphoreType.DMA,
            pltpu.SemaphoreType.DMA,
        ],
        compiler_params=pltpu.CompilerParams(use_tc_tiling_on_sc=False),
    )


@functools.lru_cache(maxsize=None)
def _sc_scatter(with_w):
    out_type = [jax.ShapeDtypeStruct((NC, NP, H), jnp.float32)]
    scratch = [
        pltpu.VMEM((1, CH), jnp.int32),
        pltpu.VMEM((1, CH), jnp.int32),
        pltpu.VMEM((CH, H), jnp.float32),
        pltpu.VMEM((CH, H), jnp.float32),
    ]
    if with_w:
        out_type.append(jax.ShapeDtypeStruct((NC, NP, 16), jnp.float32))
        scratch.append(pltpu.VMEM((CH, 16), jnp.float32))
        scratch.append(pltpu.VMEM((CH, 16), jnp.float32))
    scratch.append(pltpu.VMEM_SHARED((NP, H), jnp.float32))
    if with_w:
        scratch.append(pltpu.VMEM_SHARED((NP, 16), jnp.float32))
    scratch.append(pltpu.SemaphoreType.DMA)
    scratch.append(pltpu.SemaphoreType.DMA)
    return pl.kernel(
        functools.partial(_sc_scatter_body, with_w),
        mesh=_sc_mesh(),
        out_type=out_type,
        scratch_types=scratch,
        compiler_params=pltpu.CompilerParams(use_tc_tiling_on_sc=False),
    )


def _edge_body(with_w, hps, hpd, ea,
               whd, whs, wea, wd2, be1, we2, be2, wx, bx, *outs):
    sv = hps[...]
    dv = hpd[...]
    diff = dv - sv
    col = lax.broadcasted_iota(jnp.int32, diff.shape, 1)
    pdiff = jnp.where(col >= D, diff, 0.0)
    dist2 = jnp.sum(pdiff * pdiff, axis=1, keepdims=True)
    a = jnp.dot(dv, whd[...], preferred_element_type=jnp.float32)
    a = a + jnp.dot(sv, whs[...], preferred_element_type=jnp.float32)
    a = a + jnp.dot(ea[...], wea[...], preferred_element_type=jnp.float32)
    a = a + dist2 * wd2[...] + be1[...]
    m1 = a * jax.nn.sigmoid(a)
    b = jnp.dot(m1, we2[...], preferred_element_type=jnp.float32) + be2[...]
    m2 = b * jax.nn.sigmoid(b)
    outs[0][...] = m2
    if with_w:
        wv = jnp.tanh(jnp.sum(m2 * wx[...], axis=1, keepdims=True)
                      + bx[...][:, :1])
        wd = diff[:, D:DP] * wv
        col16 = lax.broadcasted_iota(jnp.int32, wd.shape, 1)
        outs[1][...] = jnp.where(col16 == 3, 1.0, wd)


def _edge_call(with_w):
    w_specs = [
        pl.BlockSpec((DP, H), lambda i: (0, 0)),
        pl.BlockSpec((DP, H), lambda i: (0, 0)),
        pl.BlockSpec((16, H), lambda i: (0, 0)),
        pl.BlockSpec((1, H), lambda i: (0, 0)),
        pl.BlockSpec((1, H), lambda i: (0, 0)),
        pl.BlockSpec((H, H), lambda i: (0, 0)),
        pl.BlockSpec((1, H), lambda i: (0, 0)),
        pl.BlockSpec((1, H), lambda i: (0, 0)),
        pl.BlockSpec((1, H), lambda i: (0, 0)),
    ]
    out_shape = [jax.ShapeDtypeStruct((E, H), jnp.float32)]
    out_specs = [pl.BlockSpec((BE, H), lambda i: (i, 0))]
    if with_w:
        out_shape.append(jax.ShapeDtypeStruct((E, 16), jnp.float32))
        out_specs.append(pl.BlockSpec((BE, 16), lambda i: (i, 0)))
    return pl.pallas_call(
        functools.partial(_edge_body, with_w),
        grid=(GE,),
        in_specs=[
            pl.BlockSpec((BE, DP), lambda i: (i, 0)),
            pl.BlockSpec((BE, DP), lambda i: (i, 0)),
            pl.BlockSpec((BE, 16), lambda i: (i, 0)),
        ] + w_specs,
        out_specs=out_specs,
        out_shape=out_shape,
        compiler_params=pltpu.CompilerParams(
            dimension_semantics=("arbitrary",)),
    )


def _node_body(with_c, xb, m0, m1, a0, a1, pb,
               wh1a, wh1b, bh1, wh2, bh2, *outs):
    magg = m0[...] + m1[...]
    t = (jnp.dot(xb[...], wh1a[...], preferred_element_type=jnp.float32)
         + jnp.dot(magg, wh1b[...], preferred_element_type=jnp.float32)
         + bh1[...])
    t = t * jax.nn.sigmoid(t)
    dh = jnp.dot(t, wh2[...], preferred_element_type=jnp.float32) + bh2[...]
    outs[0][...] = xb[...] + dh
    if with_c:
        ax = a0[...] + a1[...]
        cnt = jnp.maximum(ax[:, 3:4], 1.0)
        upd = ax / cnt
        col = lax.broadcasted_iota(jnp.int32, upd.shape, 1)
        outs[1][...] = pb[...] + jnp.where(col < 3, upd, 0.0)


def _node_call(with_c):
    in_specs = [
        pl.BlockSpec((BN, D), lambda i: (i, 0)),
        pl.BlockSpec((BN, H), lambda i: (i, 0)),
        pl.BlockSpec((BN, H), lambda i: (i, 0)),
        pl.BlockSpec((BN, 16), lambda i: (i, 0)),
        pl.BlockSpec((BN, 16), lambda i: (i, 0)),
        pl.BlockSpec((BN, 16), lambda i: (i, 0)),
        pl.BlockSpec((D, H), lambda i: (0, 0)),
        pl.BlockSpec((H, H), lambda i: (0, 0)),
        pl.BlockSpec((1, H), lambda i: (0, 0)),
        pl.BlockSpec((H, D), lambda i: (0, 0)),
        pl.BlockSpec((1, D), lambda i: (0, 0)),
    ]
    out_shape = [jax.ShapeDtypeStruct((N, D), jnp.float32)]
    out_specs = [pl.BlockSpec((BN, D), lambda i: (i, 0))]
    if with_c:
        out_shape.append(jax.ShapeDtypeStruct((N, 16), jnp.float32))
        out_specs.append(pl.BlockSpec((BN, 16), lambda i: (i, 0)))
    return pl.pallas_call(
        functools.partial(_node_body, with_c),
        grid=(GN,),
        in_specs=in_specs,
        out_specs=out_specs,
        out_shape=out_shape,
        compiler_params=pltpu.CompilerParams(
            dimension_semantics=("arbitrary",)),
    )


def _layer_weights(lp):
    we1 = lp["We1"]
    zpad = jnp.zeros((16, H), jnp.float32)
    return dict(
        whd=jnp.concatenate([we1[0:D], zpad], axis=0),
        whs=jnp.concatenate([we1[D:2 * D], zpad], axis=0),
        wd2=we1[2 * D:2 * D + 1],
        wea=jnp.pad(we1[2 * D + 1:], ((0, 1), (0, 0))),
        be1=lp["be1"][None, :],
        we2=lp["We2"],
        be2=lp["be2"][None, :],
        wx=lp["Wx"].T,
        bx=jnp.broadcast_to(lp["bx"].reshape(1, 1), (1, H)),
        wh1a=lp["Wh1"][0:D],
        wh1b=lp["Wh1"][D:],
        bh1=lp["bh1"][None, :],
        wh2=lp["Wh2"],
        bh2=lp["bh2"][None, :],
    )


def kernel(x, pos, edge_index, edge_attr, params):
    src3d = edge_index[0].reshape(NW, NCH, CH)
    dst3d = edge_index[1].reshape(NW, NCH, CH)
    pos_pad = jnp.pad(pos, ((0, 0), (0, 13)))
    tab0 = jnp.concatenate([x, pos_pad], axis=1)
    ea_pad = jnp.pad(edge_attr, ((0, 0), (0, 1)))
    zm = jnp.zeros((RPT, H), jnp.float32)
    za = jnp.zeros((RPT, 16), jnp.float32)
    w1 = _layer_weights(params["layers"][0])
    w2 = _layer_weights(params["layers"][1])

    hps, hpd = _sc_gather()(tab0, src3d, dst3d)
    m2, wd = _edge_call(True)(
        hps, hpd, ea_pad,
        w1["whd"], w1["whs"], w1["wea"], w1["wd2"], w1["be1"],
        w1["we2"], w1["be2"], w1["wx"], w1["bx"])
    mparts, aparts = _sc_scatter(True)(m2, wd, dst3d, zm, za)
    h1, c1 = _node_call(True)(
        x, mparts[0], mparts[1], aparts[0], aparts[1], pos_pad,
        w1["wh1a"], w1["wh1b"], w1["bh1"], w1["wh2"], w1["bh2"])
    tab1 = jnp.concatenate([h1, c1], axis=1)

    hps2, hpd2 = _sc_gather()(tab1, src3d, dst3d)
    m2b, = _edge_call(False)(
        hps2, hpd2, ea_pad,
        w2["whd"], w2["whs"], w2["wea"], w2["wd2"], w2["be1"],
        w2["we2"], w2["be2"], w2["wx"], w2["bx"])
    mparts2, = _sc_scatter(False)(m2b, dst3d, zm)
    h2, = _node_call(False)(
        h1, mparts2[0], mparts2[1], aparts[0], aparts[1], pos_pad,
        w2["wh1a"], w2["wh1b"], w2["bh1"], w2["wh2"], w2["bh2"])
    return h2


# speedup vs baseline: 6.7264x; 1.3045x over previous
import functools

import jax
import jax.numpy as jnp
from jax import lax
from jax.experimental import pallas as pl
from jax.experimental.pallas import tpu as pltpu
from jax.experimental.pallas import tpu_sc as plsc

N = 10000
E = 640000
D = 64
DT = 128
H = 128
DE = 15
NC = 2
NS = 16
NW = NC * NS
EPW = E // NW
CH = 80
NCH = EPW // CH
NP = 10240
RPT = NP // NS
BE = 1024
GE = E // BE
BN = 1000
GN = N // BN


def _sc_gather_body(tab_hbm, src_hbm, dst_hbm, hps_out, hpd_out,
                    idx_s, idx_d, bs0, bd0, bs1, bd1, sem0, sem1):
    c = lax.axis_index("c")
    s = lax.axis_index("s")
    wid = s * NC + c
    pltpu.sync_copy(src_hbm.at[wid], idx_s)
    pltpu.sync_copy(dst_hbm.at[wid], idx_d)
    bufs = ((bs0, bd0, sem0), (bs1, bd1, sem1))

    def start(g, b):
        bs, bd, sem = bufs[b]
        pltpu.make_async_copy(tab_hbm.at[idx_s.at[g]], bs, sem).start()
        pltpu.make_async_copy(tab_hbm.at[idx_d.at[g]], bd, sem).start()

    def finish(g, b):
        bs, bd, sem = bufs[b]
        pltpu.make_async_copy(tab_hbm.at[idx_s.at[g]], bs, sem).wait()
        pltpu.make_async_copy(tab_hbm.at[idx_d.at[g]], bd, sem).wait()
        e0 = wid * EPW + g * CH
        pltpu.sync_copy(bs, hps_out.at[pl.ds(e0, CH)])
        pltpu.sync_copy(bd, hpd_out.at[pl.ds(e0, CH)])

    start(0, 0)

    def body(i, carry):
        g0 = 2 * i
        start(g0 + 1, 1)
        finish(g0, 0)

        @pl.when(i < NCH // 2 - 1)
        def _():
            start(g0 + 2, 0)

        finish(g0 + 1, 1)
        return carry

    lax.fori_loop(0, NCH // 2, body, 0)


def _sc_scatter_body(with_w, *refs):
    if with_w:
        (m2_hbm, wd_hbm, dst_hbm, zm_hbm, za_hbm, outm, outa,
         idx0, idx1, m0, m1, w0, macc, aacc, sem0, sem1) = refs
        bufs = ((idx0, m0, w0, sem0), (idx1, m1, w0, sem1))
    else:
        (m2_hbm, dst_hbm, zm_hbm, outm, idx0, idx1, m0, m1,
         macc, sem0, sem1) = refs
        bufs = ((idx0, m0, None, sem0), (idx1, m1, None, sem1))
    c = lax.axis_index("c")
    s = lax.axis_index("s")
    wid = s * NC + c
    pltpu.sync_copy(zm_hbm, macc.at[pl.ds(s * RPT, RPT)])
    if with_w:
        pltpu.sync_copy(za_hbm, aacc.at[pl.ds(s * RPT, RPT)])
    plsc.subcore_barrier()

    def start(g, b):
        iv, mv, wv, sem = bufs[b]
        e0 = wid * EPW + g * CH
        pltpu.make_async_copy(dst_hbm.at[wid].at[pl.ds(g, 1)], iv, sem).start()
        pltpu.make_async_copy(m2_hbm.at[pl.ds(e0, CH)], mv, sem).start()

    def finish(g, b):
        iv, mv, wv, sem = bufs[b]
        e0 = wid * EPW + g * CH
        pltpu.make_async_copy(dst_hbm.at[wid].at[pl.ds(g, 1)], iv, sem).wait()
        pltpu.make_async_copy(m2_hbm.at[pl.ds(e0, CH)], mv, sem).wait()
        pltpu.sync_copy(mv, macc.at[iv.at[0]], add=True)
        if with_w:
            pltpu.sync_copy(wd_hbm.at[pl.ds(e0, CH)], wv)
            pltpu.sync_copy(wv, aacc.at[iv.at[0]], add=True)

    start(0, 0)

    def body(i, carry):
        g0 = 2 * i
        start(g0 + 1, 1)
        finish(g0, 0)

        @pl.when(i < NCH // 2 - 1)
        def _():
            start(g0 + 2, 0)

        finish(g0 + 1, 1)
        return carry

    lax.fori_loop(0, NCH // 2, body, 0)
    plsc.subcore_barrier()
    pltpu.sync_copy(macc.at[pl.ds(s * RPT, RPT)],
                    outm.at[c].at[pl.ds(s * RPT, RPT)])
    if with_w:
        pltpu.sync_copy(aacc.at[pl.ds(s * RPT, RPT)],
                        outa.at[c].at[pl.ds(s * RPT, RPT)])


@functools.lru_cache(maxsize=None)
def _sc_mesh():
    return plsc.VectorSubcoreMesh(core_axis_name="c", subcore_axis_name="s")


@functools.lru_cache(maxsize=None)
def _sc_gather():
    return pl.kernel(
        _sc_gather_body,
        mesh=_sc_mesh(),
        out_type=[
            jax.ShapeDtypeStruct((E, DT), jnp.float32),
            jax.ShapeDtypeStruct((E, DT), jnp.float32),
        ],
        scratch_types=[
            pltpu.VMEM((NCH, CH), jnp.int32),
            pltpu.VMEM((NCH, CH), jnp.int32),
            pltpu.VMEM((CH, DT), jnp.float32),
            pltpu.VMEM((CH, DT), jnp.float32),
            pltpu.VMEM((CH, DT), jnp.float32),
            pltpu.VMEM((CH, DT), jnp.float32),
            pltpu.SemaphoreType.DMA,
            pltpu.SemaphoreType.DMA,
        ],
        compiler_params=pltpu.CompilerParams(use_tc_tiling_on_sc=False),
    )


@functools.lru_cache(maxsize=None)
def _sc_scatter(with_w):
    out_type = [jax.ShapeDtypeStruct((NC, NP, H), jnp.float32)]
    scratch = [
        pltpu.VMEM((1, CH), jnp.int32),
        pltpu.VMEM((1, CH), jnp.int32),
        pltpu.VMEM((CH, H), jnp.float32),
        pltpu.VMEM((CH, H), jnp.float32),
    ]
    if with_w:
        out_type.append(jax.ShapeDtypeStruct((NC, NP, 16), jnp.float32))
        scratch.append(pltpu.VMEM((CH, 16), jnp.float32))
    scratch.append(pltpu.VMEM_SHARED((NP, H), jnp.float32))
    if with_w:
        scratch.append(pltpu.VMEM_SHARED((NP, 16), jnp.float32))
    scratch.append(pltpu.SemaphoreType.DMA)
    scratch.append(pltpu.SemaphoreType.DMA)
    return pl.kernel(
        functools.partial(_sc_scatter_body, with_w),
        mesh=_sc_mesh(),
        out_type=out_type,
        scratch_types=scratch,
        compiler_params=pltpu.CompilerParams(use_tc_tiling_on_sc=False),
    )


def _edge_body(with_w, hps, hpd, ea,
               whd, whs, wsq, wea, be1, we2, be2, wxm, bx, *outs):
    sv = hps[...]
    dv = hpd[...]
    diff = dv - sv
    a = jnp.dot(dv, whd[...], preferred_element_type=jnp.float32)
    a = a + jnp.dot(sv, whs[...], preferred_element_type=jnp.float32)
    a = a + jnp.dot(diff * diff, wsq[...], preferred_element_type=jnp.float32)
    a = a + jnp.dot(ea[...], wea[...], preferred_element_type=jnp.float32)
    a = a + be1[...]
    m1 = a * jax.nn.sigmoid(a)
    b = jnp.dot(m1, we2[...], preferred_element_type=jnp.float32) + be2[...]
    m2 = b * jax.nn.sigmoid(b)
    outs[0][...] = m2
    if with_w:
        wpre = jnp.dot(m2, wxm[...], preferred_element_type=jnp.float32)
        wv = jnp.tanh(wpre[:, :1] + bx[...][:, :1])
        wd = diff[:, D:D + 16] * wv
        col16 = lax.broadcasted_iota(jnp.int32, wd.shape, 1)
        outs[1][...] = jnp.where(col16 == 3, 1.0, wd)


def _edge_call(with_w):
    w_specs = [
        pl.BlockSpec((DT, H), lambda i: (0, 0)),
        pl.BlockSpec((DT, H), lambda i: (0, 0)),
        pl.BlockSpec((DT, H), lambda i: (0, 0)),
        pl.BlockSpec((DE, H), lambda i: (0, 0)),
        pl.BlockSpec((1, H), lambda i: (0, 0)),
        pl.BlockSpec((H, H), lambda i: (0, 0)),
        pl.BlockSpec((1, H), lambda i: (0, 0)),
        pl.BlockSpec((H, H), lambda i: (0, 0)),
        pl.BlockSpec((1, H), lambda i: (0, 0)),
    ]
    out_shape = [jax.ShapeDtypeStruct((E, H), jnp.float32)]
    out_specs = [pl.BlockSpec((BE, H), lambda i: (i, 0))]
    if with_w:
        out_shape.append(jax.ShapeDtypeStruct((E, 16), jnp.float32))
        out_specs.append(pl.BlockSpec((BE, 16), lambda i: (i, 0)))
    return pl.pallas_call(
        functools.partial(_edge_body, with_w),
        grid=(GE,),
        in_specs=[
            pl.BlockSpec((BE, DT), lambda i: (i, 0)),
            pl.BlockSpec((BE, DT), lambda i: (i, 0)),
            pl.BlockSpec((BE, DE), lambda i: (i, 0)),
        ] + w_specs,
        out_specs=out_specs,
        out_shape=out_shape,
        compiler_params=pltpu.CompilerParams(
            dimension_semantics=("arbitrary",)),
    )


def _node_body(with_c, tab, m0, m1, a0, a1,
               wh1a, wh1b, bh1, wh2, bh2, *outs):
    tv = tab[...]
    xb = tv[:, :D]
    magg = m0[...] + m1[...]
    t = (jnp.dot(xb, wh1a[...], preferred_element_type=jnp.float32)
         + jnp.dot(magg, wh1b[...], preferred_element_type=jnp.float32)
         + bh1[...])
    t = t * jax.nn.sigmoid(t)
    hn = xb + jnp.dot(t, wh2[...], preferred_element_type=jnp.float32) + bh2[...]
    if with_c:
        ax = a0[...] + a1[...]
        cnt = jnp.maximum(ax[:, 3:4], 1.0)
        upd = ax / cnt
        col = lax.broadcasted_iota(jnp.int32, upd.shape, 1)
        cn = tv[:, D:D + 16] + jnp.where(col < 3, upd, 0.0)
        outs[0][...] = jnp.concatenate(
            [hn, cn, jnp.zeros((hn.shape[0], DT - D - 16), jnp.float32)],
            axis=1)
    else:
        outs[0][...] = hn


def _node_call(with_c):
    in_specs = [
        pl.BlockSpec((BN, DT), lambda i: (i, 0)),
        pl.BlockSpec((BN, H), lambda i: (i, 0)),
        pl.BlockSpec((BN, H), lambda i: (i, 0)),
        pl.BlockSpec((BN, 16), lambda i: (i, 0)),
        pl.BlockSpec((BN, 16), lambda i: (i, 0)),
        pl.BlockSpec((D, H), lambda i: (0, 0)),
        pl.BlockSpec((H, H), lambda i: (0, 0)),
        pl.BlockSpec((1, H), lambda i: (0, 0)),
        pl.BlockSpec((H, D), lambda i: (0, 0)),
        pl.BlockSpec((1, D), lambda i: (0, 0)),
    ]
    if with_c:
        out_shape = [jax.ShapeDtypeStruct((N, DT), jnp.float32)]
        out_specs = [pl.BlockSpec((BN, DT), lambda i: (i, 0))]
    else:
        out_shape = [jax.ShapeDtypeStruct((N, D), jnp.float32)]
        out_specs = [pl.BlockSpec((BN, D), lambda i: (i, 0))]
    return pl.pallas_call(
        functools.partial(_node_body, with_c),
        grid=(GN,),
        in_specs=in_specs,
        out_specs=out_specs,
        out_shape=out_shape,
        compiler_params=pltpu.CompilerParams(
            dimension_semantics=("arbitrary",)),
    )


def _layer_weights(lp):
    we1 = lp["We1"]
    z48 = jnp.zeros((DT - D - 16, H), jnp.float32)
    z16 = jnp.zeros((16, H), jnp.float32)
    wd2 = we1[2 * D:2 * D + 1]
    wsq = jnp.concatenate(
        [jnp.zeros((D, H), jnp.float32),
         jnp.broadcast_to(wd2, (16, H)), z48], axis=0)
    wxm = jnp.pad(lp["Wx"], ((0, 0), (0, H - 1)))
    return dict(
        whd=jnp.concatenate([we1[0:D], z16, z48], axis=0),
        whs=jnp.concatenate([we1[D:2 * D], z16, z48], axis=0),
        wsq=wsq,
        wea=we1[2 * D + 1:],
        be1=lp["be1"][None, :],
        we2=lp["We2"],
        be2=lp["be2"][None, :],
        wxm=wxm,
        bx=jnp.broadcast_to(lp["bx"].reshape(1, 1), (1, H)),
        wh1a=lp["Wh1"][0:D],
        wh1b=lp["Wh1"][D:],
        bh1=lp["bh1"][None, :],
        wh2=lp["Wh2"],
        bh2=lp["bh2"][None, :],
    )


def kernel(x, pos, edge_index, edge_attr, params):
    src = edge_index[0]
    dst = edge_index[1]
    src3d = src.reshape(NW, NCH, CH)
    dst3d = dst.reshape(NW, NCH, CH)
    tab0 = jnp.concatenate(
        [x, pos, jnp.zeros((N, DT - D - 3), jnp.float32)], axis=1)
    zm = jnp.zeros((RPT, H), jnp.float32)
    za = jnp.zeros((RPT, 16), jnp.float32)
    w1 = _layer_weights(params["layers"][0])
    w2 = _layer_weights(params["layers"][1])

    hps, hpd = _sc_gather()(tab0, src3d, dst3d)
    m2, wd = _edge_call(True)(
        hps, hpd, edge_attr,
        w1["whd"], w1["whs"], w1["wsq"], w1["wea"], w1["be1"],
        w1["we2"], w1["be2"], w1["wxm"], w1["bx"])
    mparts, aparts = _sc_scatter(True)(m2, wd, dst3d, zm, za)
    tab1, = _node_call(True)(
        tab0, mparts[0], mparts[1], aparts[0], aparts[1],
        w1["wh1a"], w1["wh1b"], w1["bh1"], w1["wh2"], w1["bh2"])

    hps2, hpd2 = _sc_gather()(tab1, src3d, dst3d)
    m2b, = _edge_call(False)(
        hps2, hpd2, edge_attr,
        w2["whd"], w2["whs"], w2["wsq"], w2["wea"], w2["be1"],
        w2["we2"], w2["be2"], w2["wxm"], w2["bx"])
    mparts2, = _sc_scatter(False)(m2b, dst3d, zm)
    h2, = _node_call(False)(
        tab1, mparts2[0], mparts2[1], aparts[0], aparts[1],
        w2["wh1a"], w2["wh1b"], w2["bh1"], w2["wh2"], w2["bh2"])
    return h2
